# Initial kernel scaffold; baseline (speedup 1.0000x reference)
import jax
import jax.numpy as jnp
from jax.experimental import pallas as pl


def kernel(feat, edge_index, edge_weight, W, b):
    raise NotImplementedError("write your pallas kernel here")


# speedup vs baseline: 23.4755x; 23.4755x over previous
import functools

import jax
import jax.numpy as jnp
from jax import lax
from jax.experimental import pallas as pl
from jax.experimental.pallas import tpu as pltpu
from jax.experimental.pallas import tpu_sc as plsc

N = 10000
E = 320000
D = 128
NC = 2
NS = 16
L = 16
NW = NC * NS
EPW = E // NW
B = 80
NB = EPW // B
NPAD = 10240
ZCH = NPAD // NS
RB = 1000
GRID = N // RB

_mesh = plsc.VectorSubcoreMesh(
    core_axis_name="c", subcore_axis_name="s", num_cores=NC, num_subcores=NS)


def _zero16():
    return jnp.zeros((L,), jnp.float32)


def _deg_body(dst_hbm, ew_hbm, degp_hbm, dst_v, ew_v, zbuf, deg_sh):
    cid = lax.axis_index("c")
    sid = lax.axis_index("s")
    wid = sid * NC + cid
    pltpu.sync_copy(dst_hbm.at[wid], dst_v)
    pltpu.sync_copy(ew_hbm.at[pl.ds(wid * EPW, EPW)], ew_v)

    def zb(t, carry):
        zbuf[pl.ds(t * L, L)] = _zero16()
        return carry
    lax.fori_loop(0, ZCH // L, zb, 0)
    pltpu.sync_copy(zbuf, deg_sh.at[pl.ds(sid * ZCH, ZCH)])
    plsc.subcore_barrier()

    def body(j, carry):
        pltpu.sync_copy(ew_v.at[pl.ds(j * B, B)], deg_sh.at[dst_v.at[j]],
                        add=True)
        return carry
    lax.fori_loop(0, NB, body, 0)
    plsc.subcore_barrier()
    pltpu.sync_copy(deg_sh.at[pl.ds(sid * ZCH, ZCH)],
                    degp_hbm.at[cid, pl.ds(sid * ZCH, ZCH)])


_deg_call = pl.kernel(
    _deg_body,
    out_type=jax.ShapeDtypeStruct((NC, NPAD), jnp.float32),
    mesh=_mesh,
    scratch_types=[
        pltpu.VMEM((NB, B), jnp.int32),
        pltpu.VMEM((EPW,), jnp.float32),
        pltpu.VMEM((ZCH,), jnp.float32),
        pltpu.VMEM_SHARED((NPAD,), jnp.float32),
    ],
)


def _agg_body(src_hbm, dst_hbm, ew_hbm, y_hbm, accp_hbm,
              src_v, dst_v, ew_v, rows, acc_sh, gsem):
    cid = lax.axis_index("c")
    sid = lax.axis_index("s")
    wid = sid * NC + cid
    pltpu.sync_copy(src_hbm.at[pl.ds(wid * EPW, EPW)], src_v)
    pltpu.sync_copy(dst_hbm.at[wid], dst_v)
    pltpu.sync_copy(ew_hbm.at[pl.ds(wid * EPW, EPW)], ew_v)

    def zb(t, carry):
        for k in range(D // L):
            rows[t, pl.ds(k * L, L)] = _zero16()
        return carry
    lax.fori_loop(0, B, zb, 0)
    for z in range(ZCH // B):
        pltpu.sync_copy(rows, acc_sh.at[pl.ds(sid * ZCH + z * B, B)])
    plsc.subcore_barrier()

    def body(j, carry):
        pltpu.async_copy(y_hbm.at[src_v.at[pl.ds(j * B, B)]], rows,
                         gsem).wait()

        def scale(g, c2):
            wvec = ew_v[pl.ds(j * B + g * L, L)]
            for i in range(L):
                wv = lax.broadcast_in_dim(wvec[i], (L,), ())
                e = g * L + i
                for k in range(D // L):
                    sl = pl.ds(k * L, L)
                    rows[e, sl] = rows[e, sl] * wv
            return c2
        lax.fori_loop(0, B // L, scale, 0)
        pltpu.sync_copy(rows, acc_sh.at[dst_v.at[j]], add=True)
        return carry
    lax.fori_loop(0, NB, body, 0)
    plsc.subcore_barrier()
    pltpu.sync_copy(acc_sh.at[pl.ds(sid * ZCH, ZCH)],
                    accp_hbm.at[cid, pl.ds(sid * ZCH, ZCH)])


_agg_call = pl.kernel(
    _agg_body,
    out_type=jax.ShapeDtypeStruct((NC, NPAD, D), jnp.float32),
    mesh=_mesh,
    scratch_types=[
        pltpu.VMEM((EPW,), jnp.int32),
        pltpu.VMEM((NB, B), jnp.int32),
        pltpu.VMEM((EPW,), jnp.float32),
        pltpu.VMEM((B, D), jnp.float32),
        pltpu.VMEM_SHARED((NPAD, D), jnp.float32),
        pltpu.SemaphoreType.DMA,
    ],
)


def _lin_body(feat_ref, w_ref, degt_ref, y_ref):
    x = jnp.dot(feat_ref[...], w_ref[...], preferred_element_type=jnp.float32)
    d = degt_ref[...]
    dis = lax.rsqrt(d[:, 0:1] + d[:, 1:2] + 1.0)
    y_ref[...] = x * dis


_lin_call = pl.pallas_call(
    _lin_body,
    grid=(GRID,),
    in_specs=[
        pl.BlockSpec((RB, D), lambda i: (i, 0)),
        pl.BlockSpec((D, D), lambda i: (0, 0)),
        pl.BlockSpec((RB, NC), lambda i: (i, 0)),
    ],
    out_specs=pl.BlockSpec((RB, D), lambda i: (i, 0)),
    out_shape=jax.ShapeDtypeStruct((N, D), jnp.float32),
)


def _fin_body(accp_ref, y_ref, degt_ref, b_ref, out_ref):
    d = degt_ref[...]
    dis = lax.rsqrt(d[:, 0:1] + d[:, 1:2] + 1.0)
    s = accp_ref[0] + accp_ref[1] + y_ref[...]
    out_ref[...] = jnp.tanh(dis * s + b_ref[...][None, :])


_fin_call = pl.pallas_call(
    _fin_body,
    grid=(GRID,),
    in_specs=[
        pl.BlockSpec((NC, RB, D), lambda i: (0, i, 0)),
        pl.BlockSpec((RB, D), lambda i: (i, 0)),
        pl.BlockSpec((RB, NC), lambda i: (i, 0)),
        pl.BlockSpec((D,), lambda i: (0,)),
    ],
    out_specs=pl.BlockSpec((RB, D), lambda i: (i, 0)),
    out_shape=jax.ShapeDtypeStruct((N, D), jnp.float32),
)


def kernel(feat, edge_index, edge_weight, W, b):
    src = edge_index[0].astype(jnp.int32)
    dst = edge_index[1].astype(jnp.int32)
    ew = edge_weight.astype(jnp.float32)
    dst3 = dst.reshape(NW, NB, B)
    degp = _deg_call(dst3, ew)
    degt = jnp.transpose(degp)
    y = _lin_call(feat, W, degt)
    accp = _agg_call(src, dst3, ew, y)
    return _fin_call(accp, y, degt, b)
